# baseline (speedup 1.0000x reference)
import functools
import math

import jax
import jax.numpy as jnp
from jax import lax
from jax.experimental import pallas as pl
from jax.experimental.pallas import tpu as pltpu

_EMBED = 256
_HEADS = 4
_LEVELS = 4
_POINTS = 8
_DDIM = 32
_NZ = 4
_DH = _EMBED // _HEADS
_SHAPES = ((92, 160), (46, 80), (23, 40), (12, 20))
_STARTS = (0, 14720, 18400, 19320)
_NV = 19560
_QB = 128
_LP = _LEVELS * _POINTS


def _vproj_kernel(v_ref, w_ref, b_ref, o_ref):
    o_ref[...] = jnp.dot(v_ref[...], w_ref[...],
                         preferred_element_type=jnp.float32) + b_ref[0, :][None, :]


def _vproj(value2, W_val, b_val2):
    rb = 4096
    grid = (pl.cdiv(_NV, rb),)
    return pl.pallas_call(
        _vproj_kernel,
        grid=grid,
        in_specs=[
            pl.BlockSpec((rb, _EMBED), lambda r: (r, 0)),
            pl.BlockSpec((_EMBED, _EMBED), lambda r: (0, 0)),
            pl.BlockSpec((1, _EMBED), lambda r: (0, 0)),
        ],
        out_specs=pl.BlockSpec((rb, _EMBED), lambda r: (r, 0)),
        out_shape=jax.ShapeDtypeStruct((_NV, _EMBED), jnp.float32),
    )(value2, W_val, b_val2)


def _main_kernel(q_ref, rp_ref, vt_ref, dist_ref,
                 woff_ref, boff_ref, wattn_ref, battn_ref,
                 woffd_ref, boffd_ref, o_ref,
                 aw_s, idx_s, w_s):
    q = q_ref[...]
    offs = jnp.dot(q, woff_ref[0], preferred_element_type=jnp.float32)
    offs = offs + boff_ref[0]
    offd = jnp.dot(q, woffd_ref[0], preferred_element_type=jnp.float32)
    offd = offd + boffd_ref[0]
    lg = jnp.dot(q, wattn_ref[0], preferred_element_type=jnp.float32)
    lg = lg + battn_ref[0]
    lg = lg - jnp.max(lg, axis=-1, keepdims=True)
    e = jnp.exp(lg)
    aw_s[...] = e / jnp.sum(e, axis=-1, keepdims=True)

    o_ref[...] = jnp.zeros_like(o_ref)

    for l in range(_LEVELS):
        Hl, Wl = _SHAPES[l]
        s0 = _STARTS[l]
        fH = float(Hl)
        fW = float(Wl)
        fD = float(_DDIM)
        for p in range(_POINTS):
            lp = l * _POINTS + p
            nz = p % _NZ
            u = offs[:, lp * 2:lp * 2 + 1]
            v = offs[:, lp * 2 + 1:lp * 2 + 2]
            d = offd[:, lp:lp + 1]
            rx = rp_ref[:, nz * 3:nz * 3 + 1]
            ry = rp_ref[:, nz * 3 + 1:nz * 3 + 2]
            rz = rp_ref[:, nz * 3 + 2:nz * 3 + 3]
            x = (rx + u / fW) * fW - 0.5
            y = (ry + v / fH) * fH - 0.5
            z = (rz + d / fD) * fD - 0.5
            x0 = jnp.floor(x)
            y0 = jnp.floor(y)
            z0 = jnp.floor(z)
            tx = x - x0
            ty = y - y0
            tz = z - z0
            corners = ((y0, x0, (1.0 - ty) * (1.0 - tx)),
                       (y0, x0 + 1.0, (1.0 - ty) * tx),
                       (y0 + 1.0, x0, ty * (1.0 - tx)),
                       (y0 + 1.0, x0 + 1.0, ty * tx))
            for c, (cy, cx, wgt) in enumerate(corners):
                m = ((cy >= 0.0) & (cy <= fH - 1.0)
                     & (cx >= 0.0) & (cx <= fW - 1.0))
                iy = jnp.clip(cy, 0.0, fH - 1.0).astype(jnp.int32)
                ix = jnp.clip(cx, 0.0, fW - 1.0).astype(jnp.int32)
                idx = s0 + iy * Wl + ix
                idx_s[c, :, lp:lp + 1] = idx
                w_s[c, :, lp:lp + 1] = wgt * m.astype(jnp.float32)
            mz0 = (z0 >= 0.0) & (z0 <= fD - 1.0)
            mz1 = (z0 + 1.0 >= 0.0) & (z0 + 1.0 <= fD - 1.0)
            iz0 = jnp.clip(z0, 0.0, fD - 1.0).astype(jnp.int32)
            iz1 = jnp.clip(z0 + 1.0, 0.0, fD - 1.0).astype(jnp.int32)
            idx_s[4, :, lp:lp + 1] = iz0
            idx_s[5, :, lp:lp + 1] = iz1
            w_s[4, :, lp:lp + 1] = (1.0 - tz) * mz0.astype(jnp.float32)
            w_s[5, :, lp:lp + 1] = tz * mz1.astype(jnp.float32)

    iota32 = lax.broadcasted_iota(jnp.int32, (1, _DDIM), 1)

    def body(i, carry):
        acc = jnp.zeros((1, _DH), jnp.float32)
        for lp in range(_LP):
            a = aw_s[i, lp]
            iz0 = idx_s[4, i, lp]
            iz1 = idx_s[5, i, lp]
            wz0 = w_s[4, i, lp]
            wz1 = w_s[5, i, lp]
            zsel = (jnp.where(iota32 == iz0, wz0, 0.0)
                    + jnp.where(iota32 == iz1, wz1, 0.0))
            samp = jnp.zeros((1, _DH), jnp.float32)
            ds = jnp.float32(0.0)
            for c in range(4):
                r = idx_s[c, i, lp]
                w2 = w_s[c, i, lp]
                samp = samp + w2 * vt_ref[0, pl.ds(r, 1), :]
                drow = dist_ref[pl.ds(r, 1), :]
                ds = ds + w2 * jnp.sum(drow * zsel)
            acc = acc + (a * ds) * samp
        o_ref[0, pl.ds(i, 1), :] = acc
        return carry

    lax.fori_loop(0, _QB, body, jnp.int32(0))


def kernel(query, value, value_dpt_dist, reference_points, spatial_shapes,
           level_start_index, W_off, b_off, W_attn, b_attn, W_val, b_val,
           W_offd, b_offd):
    bs, nq, E = query.shape
    q2 = query.reshape(nq, E)
    rp2 = reference_points.reshape(nq, _NZ * 3)
    value2 = value.reshape(_NV, E)
    dist2 = value_dpt_dist.reshape(_NV, _DDIM)
    vproj = _vproj(value2, W_val, b_val.reshape(1, E))
    vt = vproj.reshape(_NV, _HEADS, _DH).transpose(1, 0, 2)
    woff3 = W_off.reshape(E, _HEADS, _LP * 2).transpose(1, 0, 2)
    wattn3 = W_attn.reshape(E, _HEADS, _LP).transpose(1, 0, 2)
    woffd3 = W_offd.reshape(E, _HEADS, _LP).transpose(1, 0, 2)
    boff3 = b_off.reshape(_HEADS, 1, _LP * 2)
    battn3 = b_attn.reshape(_HEADS, 1, _LP)
    boffd3 = b_offd.reshape(_HEADS, 1, _LP)

    nqb = pl.cdiv(nq, _QB)
    out = pl.pallas_call(
        _main_kernel,
        grid=(_HEADS, nqb),
        in_specs=[
            pl.BlockSpec((_QB, E), lambda h, qb: (qb, 0)),
            pl.BlockSpec((_QB, _NZ * 3), lambda h, qb: (qb, 0)),
            pl.BlockSpec((1, _NV, _DH), lambda h, qb: (h, 0, 0)),
            pl.BlockSpec((_NV, _DDIM), lambda h, qb: (0, 0)),
            pl.BlockSpec((1, E, _LP * 2), lambda h, qb: (h, 0, 0)),
            pl.BlockSpec((1, 1, _LP * 2), lambda h, qb: (h, 0, 0)),
            pl.BlockSpec((1, E, _LP), lambda h, qb: (h, 0, 0)),
            pl.BlockSpec((1, 1, _LP), lambda h, qb: (h, 0, 0)),
            pl.BlockSpec((1, E, _LP), lambda h, qb: (h, 0, 0)),
            pl.BlockSpec((1, 1, _LP), lambda h, qb: (h, 0, 0)),
        ],
        out_specs=pl.BlockSpec((1, _QB, _DH), lambda h, qb: (h, qb, 0)),
        out_shape=jax.ShapeDtypeStruct((_HEADS, nq, _DH), jnp.float32),
        scratch_shapes=[
            pltpu.VMEM((_QB, _LP), jnp.float32),
            pltpu.VMEM((6, _QB, _LP), jnp.int32),
            pltpu.VMEM((6, _QB, _LP), jnp.float32),
        ],
    )(q2, rp2, vt, dist2,
      woff3, boff3, wattn3, battn3, woffd3, boffd3)

    return out.transpose(1, 0, 2).reshape(bs, nq, E)
